# baseline (speedup 1.0000x reference)
import functools

import jax
import jax.numpy as jnp
from jax import lax
from jax.experimental import pallas as pl
from jax.experimental.pallas import tpu as pltpu
from jax.experimental.pallas import tpu_sc as plsc

VOCAB = 1000000
EMBED_DIM = 16
BATCH = 16384
FIELDS = 26
N = BATCH * FIELDS
NUM_CORES = 2
NUM_SUBCORES = 16
NW = NUM_CORES * NUM_SUBCORES
GW = 128
G = N // GW
G_PER_W = G // NW
K = 13
CHUNK = K * GW
NCHUNK = G_PER_W // K
L = 16


def _make_kernel():
    mesh = plsc.VectorSubcoreMesh(core_axis_name="c", subcore_axis_name="s")

    @functools.partial(
        pl.kernel,
        mesh=mesh,
        out_type=jax.ShapeDtypeStruct((EMBED_DIM, N), jnp.float32),
        scratch_types=[
            pltpu.VMEM((G_PER_W, GW), jnp.int32),
            pltpu.VMEM((2, CHUNK, EMBED_DIM), jnp.float32),
            pltpu.VMEM((2, EMBED_DIM, CHUNK), jnp.float32),
            pltpu.SemaphoreType.DMA,
            pltpu.SemaphoreType.DMA,
        ],
        compiler_params=pltpu.CompilerParams(
            use_tc_tiling_on_sc=False, needs_layout_passes=False),
    )
    def k(idx_hbm, table_hbm, out_hbm, idx_v, rows_v, planes_v, gsem, wsem):
        wid = lax.axis_index("s") * NUM_CORES + lax.axis_index("c")
        gbase = wid * G_PER_W
        fbase = gbase * GW
        pltpu.sync_copy(idx_hbm.at[pl.ds(gbase, G_PER_W)], idx_v)

        def fire(c, buf):
            for j in range(K):
                pltpu.async_copy(
                    table_hbm.at[idx_v.at[c * K + j]],
                    rows_v.at[buf, pl.ds(j * GW, GW), :], gsem)

        def drain_gathers(buf):
            for j in range(K):
                pltpu.make_async_copy(
                    table_hbm.at[idx_v.at[j]],
                    rows_v.at[buf, pl.ds(j * GW, GW), :], gsem).wait()

        def plane_start(c, buf):
            pltpu.async_copy(
                planes_v.at[buf],
                out_hbm.at[:, pl.ds(fbase + c * CHUNK, CHUNK)], wsem)

        def plane_wait(c, buf):
            pltpu.make_async_copy(
                planes_v.at[buf],
                out_hbm.at[:, pl.ds(fbase + c * CHUNK, CHUNK)], wsem).wait()

        fire(0, 0)
        lane = lax.iota(jnp.int32, L)
        RUN = 16

        def chunk(c, carry):
            buf = lax.rem(c, 2)

            @pl.when(c + 1 < NCHUNK)
            def _():
                fire(c + 1, 1 - buf)

            drain_gathers(buf)

            @pl.when(c >= 2)
            def _():
                plane_wait(c, buf)

            rows = rows_v.at[buf]
            planes = planes_v.at[buf]

            def group(g, carry2):
                i0 = g * RUN
                for r in range(RUN):
                    v = rows[i0 + r, :]
                    plsc.store_scatter(
                        planes, [lane, jnp.full((L,), i0 + r, jnp.int32)], v)
                return carry2

            lax.fori_loop(0, CHUNK // RUN, group, 0)
            plane_start(c, buf)
            return carry

        lax.fori_loop(0, NCHUNK, chunk, 0)
        plane_wait(NCHUNK - 2, 0)
        plane_wait(NCHUNK - 1, 1)

    return k


_embed_gather = _make_kernel()


def kernel(x, table):
    idx = x.reshape(G, GW).astype(jnp.int32)
    planes = _embed_gather(idx, table)
    return planes.T.reshape(BATCH, FIELDS, EMBED_DIM)


# speedup vs baseline: 1.3981x; 1.3981x over previous
import functools

import jax
import jax.numpy as jnp
from jax import lax
from jax.experimental import pallas as pl
from jax.experimental.pallas import tpu as pltpu
from jax.experimental.pallas import tpu_sc as plsc

VOCAB = 1000000
EMBED_DIM = 16
BATCH = 16384
FIELDS = 26
N = BATCH * FIELDS
NUM_CORES = 2
NUM_SUBCORES = 16
NW = NUM_CORES * NUM_SUBCORES
GW = 128
G = N // GW
G_PER_W = G // NW
K = 13
CHUNK = K * GW
NCHUNK = G_PER_W // K
L = 16


def _make_kernel():
    mesh = plsc.VectorSubcoreMesh(core_axis_name="c", subcore_axis_name="s")

    @functools.partial(
        pl.kernel,
        mesh=mesh,
        out_type=jax.ShapeDtypeStruct((EMBED_DIM, N), jnp.float32),
        scratch_types=[
            pltpu.VMEM((G_PER_W, GW), jnp.int32),
            pltpu.VMEM((2, CHUNK, EMBED_DIM), jnp.float32),
            pltpu.VMEM((2, EMBED_DIM, CHUNK), jnp.float32),
            pltpu.SemaphoreType.DMA,
            pltpu.SemaphoreType.DMA,
        ],
        compiler_params=pltpu.CompilerParams(
            use_tc_tiling_on_sc=False, needs_layout_passes=False),
    )
    def k(idx_hbm, table_hbm, out_hbm, idx_v, rows_v, planes_v, gsem, wsem):
        wid = lax.axis_index("s") * NUM_CORES + lax.axis_index("c")
        gbase = wid * G_PER_W
        fbase = gbase * GW
        pltpu.sync_copy(idx_hbm.at[pl.ds(gbase, G_PER_W)], idx_v)

        def fire(c, buf):
            for j in range(K):
                pltpu.async_copy(
                    table_hbm.at[idx_v.at[c * K + j]],
                    rows_v.at[buf, pl.ds(j * GW, GW), :], gsem)

        def drain_gathers(buf):
            for j in range(K):
                pltpu.make_async_copy(
                    table_hbm.at[idx_v.at[j]],
                    rows_v.at[buf, pl.ds(j * GW, GW), :], gsem).wait()

        def plane_start(c, buf):
            pltpu.async_copy(
                planes_v.at[buf],
                out_hbm.at[:, pl.ds(fbase + c * CHUNK, CHUNK)], wsem)

        def plane_wait(c, buf):
            pltpu.make_async_copy(
                planes_v.at[buf],
                out_hbm.at[:, pl.ds(fbase + c * CHUNK, CHUNK)], wsem).wait()

        fire(0, 0)
        lane = lax.iota(jnp.int32, L)
        RUN = 16

        def chunk(c, carry):
            buf = lax.rem(c, 2)

            @pl.when(c + 1 < NCHUNK)
            def _():
                fire(c + 1, 1 - buf)

            drain_gathers(buf)

            @pl.when(c >= 2)
            def _():
                plane_wait(c, buf)

            rows = rows_v.at[buf]
            planes = planes_v.at[buf]

            def group(g, carry2):
                i0 = g * RUN
                for r in range(RUN):
                    v = rows[i0 + r, :]
                    plsc.store_scatter(
                        planes, [lane, jnp.full((L,), i0 + r, jnp.int32)], v)
                return carry2

            lax.fori_loop(0, CHUNK // RUN, group, 0)
            plane_start(c, buf)
            return carry

        lax.fori_loop(0, NCHUNK, chunk, 0)
        plane_wait(NCHUNK - 2, 0)
        plane_wait(NCHUNK - 1, 1)

    return k


_embed_gather = _make_kernel()


def kernel(x, table):
    idx = x.T.reshape(G, GW).astype(jnp.int32)
    planes = _embed_gather(idx, table)
    return planes.reshape(EMBED_DIM, FIELDS, BATCH).transpose(2, 1, 0)
